# baseline (speedup 1.0000x reference)
import functools

import jax
import jax.numpy as jnp
from jax.experimental import pallas as pl
from jax.experimental.pallas import tpu as pltpu

T = 2048
DM = 1024
H = 16
KVH = 8
D = 64
DFF = 2048
E = 8
TOPK = 2
EPS = 1e-05

TB = 256
FB = 512
NEG = -1e30


def _qkv_kernel(h_ref, nw_ref, w_ref, qkv_ref):
    x = h_ref[...]
    nw = nw_ref[...]
    x = x * jax.lax.rsqrt(jnp.mean(x * x, axis=-1, keepdims=True) + EPS) * nw
    qkv_ref[...] = jax.lax.dot_general(
        x, w_ref[...], (((1,), (1,)), ((), ())),
        preferred_element_type=jnp.float32)


def _qkv_call(hidden, in_norm_w2d, qkv_w):
    return pl.pallas_call(
        _qkv_kernel,
        grid=(T // TB,),
        in_specs=[
            pl.BlockSpec((TB, DM), lambda i: (i, 0)),
            pl.BlockSpec((1, DM), lambda i: (0, 0)),
            pl.BlockSpec(((H + 2 * KVH) * D, DM), lambda i: (0, 0)),
        ],
        out_specs=pl.BlockSpec((TB, (H + 2 * KVH) * D), lambda i: (i, 0)),
        out_shape=jax.ShapeDtypeStruct((T, (H + 2 * KVH) * D), jnp.float32),
    )(hidden, in_norm_w2d, qkv_w)


def _rope(x, c, s):
    half = D // 2
    x1 = x[:, :half]
    x2 = x[:, half:]
    return jnp.concatenate([x1 * c - x2 * s, x2 * c + x1 * s], axis=-1)


def _attn_kernel(q_ref, k_ref, v_ref, cq_ref, sq_ref, ck_ref, sk_ref, o_ref):
    qb = pl.program_id(1)
    q = _rope(q_ref[0], cq_ref[...], sq_ref[...]) * (D ** -0.5)
    k = _rope(k_ref[0], ck_ref[...], sk_ref[...])
    s = jax.lax.dot_general(q, k, (((1,), (1,)), ((), ())),
                            preferred_element_type=jnp.float32)
    row = qb * TB + jax.lax.broadcasted_iota(jnp.int32, (TB, T), 0)
    col = jax.lax.broadcasted_iota(jnp.int32, (TB, T), 1)
    s = jnp.where(col <= row, s, NEG)
    m = jnp.max(s, axis=-1, keepdims=True)
    p = jnp.exp(s - m)
    p = p / jnp.sum(p, axis=-1, keepdims=True)
    o_ref[0] = jnp.dot(p, v_ref[0], preferred_element_type=jnp.float32)


def _attn_call(q3, k3, v3, cos, sin):
    return pl.pallas_call(
        _attn_kernel,
        grid=(H, T // TB),
        in_specs=[
            pl.BlockSpec((1, TB, D), lambda h, i: (h, i, 0)),
            pl.BlockSpec((1, T, D), lambda h, i: (h // (H // KVH), 0, 0)),
            pl.BlockSpec((1, T, D), lambda h, i: (h // (H // KVH), 0, 0)),
            pl.BlockSpec((TB, D // 2), lambda h, i: (i, 0)),
            pl.BlockSpec((TB, D // 2), lambda h, i: (i, 0)),
            pl.BlockSpec((T, D // 2), lambda h, i: (0, 0)),
            pl.BlockSpec((T, D // 2), lambda h, i: (0, 0)),
        ],
        out_specs=pl.BlockSpec((1, TB, D), lambda h, i: (h, i, 0)),
        out_shape=jax.ShapeDtypeStruct((H, T, D), jnp.float32),
    )(q3, k3, v3, cos, sin, cos, sin)


def _post_kernel(ctx_ref, res_ref, ow_ref, nw_ref, gw_ref,
                 ares_ref, x2_ref, gates_ref):
    attn = jax.lax.dot_general(ctx_ref[...], ow_ref[...], (((1,), (1,)), ((), ())),
                               preferred_element_type=jnp.float32)
    ares = attn + res_ref[...]
    ares_ref[...] = ares
    x2 = ares * jax.lax.rsqrt(jnp.mean(ares * ares, axis=-1, keepdims=True) + EPS)
    x2 = x2 * nw_ref[...]
    x2_ref[...] = x2
    logits = jax.lax.dot_general(x2, gw_ref[...], (((1,), (1,)), ((), ())),
                                 preferred_element_type=jnp.float32)
    m = jnp.max(logits, axis=-1, keepdims=True)
    p = jnp.exp(logits - m)
    probs = p / jnp.sum(p, axis=-1, keepdims=True)
    e_iota = jax.lax.broadcasted_iota(jnp.int32, (TB, E), 1)
    i1 = jnp.argmax(probs, axis=-1, keepdims=True)
    p1 = jnp.max(probs, axis=-1, keepdims=True)
    probs2 = jnp.where(e_iota == i1, -1.0, probs)
    i2 = jnp.argmax(probs2, axis=-1, keepdims=True)
    p2 = jnp.max(probs2, axis=-1, keepdims=True)
    tot = p1 + p2
    gates_ref[...] = jnp.where(e_iota == i1, p1 / tot,
                               jnp.where(e_iota == i2, p2 / tot, 0.0))


def _post_call(ctx2d, hidden, o_w, post_norm_w2d, gate_w):
    return pl.pallas_call(
        _post_kernel,
        grid=(T // TB,),
        in_specs=[
            pl.BlockSpec((TB, H * D), lambda i: (i, 0)),
            pl.BlockSpec((TB, DM), lambda i: (i, 0)),
            pl.BlockSpec((DM, H * D), lambda i: (0, 0)),
            pl.BlockSpec((1, DM), lambda i: (0, 0)),
            pl.BlockSpec((E, DM), lambda i: (0, 0)),
        ],
        out_specs=[
            pl.BlockSpec((TB, DM), lambda i: (i, 0)),
            pl.BlockSpec((TB, DM), lambda i: (i, 0)),
            pl.BlockSpec((TB, E), lambda i: (i, 0)),
        ],
        out_shape=[
            jax.ShapeDtypeStruct((T, DM), jnp.float32),
            jax.ShapeDtypeStruct((T, DM), jnp.float32),
            jax.ShapeDtypeStruct((T, E), jnp.float32),
        ],
    )(ctx2d, hidden, o_w, post_norm_w2d, gate_w)


def _moe_kernel(x2_ref, gates_ref, w1_ref, w3_ref, w2_ref, out_ref):
    e = pl.program_id(1)
    f = pl.program_id(2)

    @pl.when(jnp.logical_and(e == 0, f == 0))
    def _():
        out_ref[...] = jnp.zeros_like(out_ref)

    x = x2_ref[...]
    g = jax.lax.dot_general(x, w1_ref[0], (((1,), (1,)), ((), ())),
                            preferred_element_type=jnp.float32)
    u = jax.lax.dot_general(x, w3_ref[0], (((1,), (1,)), ((), ())),
                            preferred_element_type=jnp.float32)
    act = (g * jax.lax.logistic(g)) * u
    y = jax.lax.dot_general(act, w2_ref[0], (((1,), (1,)), ((), ())),
                            preferred_element_type=jnp.float32)
    e_iota = jax.lax.broadcasted_iota(jnp.int32, (TB, E), 1)
    gate_col = jnp.sum(jnp.where(e_iota == e, gates_ref[...], 0.0),
                       axis=1, keepdims=True)
    out_ref[...] += gate_col * y


def _moe_call(x2, gates3, w1, w3, w2):
    return pl.pallas_call(
        _moe_kernel,
        grid=(T // TB, E, DFF // FB),
        in_specs=[
            pl.BlockSpec((TB, DM), lambda i, e, f: (i, 0)),
            pl.BlockSpec((TB, E), lambda i, e, f: (i, 0)),
            pl.BlockSpec((1, FB, DM), lambda i, e, f: (e, f, 0)),
            pl.BlockSpec((1, FB, DM), lambda i, e, f: (e, f, 0)),
            pl.BlockSpec((1, DM, FB), lambda i, e, f: (e, 0, f)),
        ],
        out_specs=pl.BlockSpec((TB, DM), lambda i, e, f: (i, 0)),
        out_shape=jax.ShapeDtypeStruct((T, DM), jnp.float32),
    )(x2, gates3, w1, w3, w2)


@jax.jit
def kernel(hidden_states, cos, sin, in_norm_w, post_norm_w, qkv_w, o_w,
           gate_w, w1, w3, w2):
    qkv = _qkv_call(hidden_states, in_norm_w.reshape(1, DM), qkv_w)
    q3 = qkv[:, :H * D].reshape(T, H, D).transpose(1, 0, 2)
    k3 = qkv[:, H * D:(H + KVH) * D].reshape(T, KVH, D).transpose(1, 0, 2)
    v3 = qkv[:, (H + KVH) * D:].reshape(T, KVH, D).transpose(1, 0, 2)
    ctx = _attn_call(q3, k3, v3, cos, sin)
    ctx2d = ctx.transpose(1, 0, 2).reshape(T, H * D)
    attn_res, x2, gates = _post_call(ctx2d, hidden_states, o_w,
                                     post_norm_w.reshape(1, DM), gate_w)
    moe_out = _moe_call(x2, gates, w1, w3, w2)
    return (moe_out, attn_res)


# speedup vs baseline: 1.1490x; 1.1490x over previous
import functools

import jax
import jax.numpy as jnp
from jax.experimental import pallas as pl
from jax.experimental.pallas import tpu as pltpu

T = 2048
DM = 1024
H = 16
KVH = 8
D = 64
DFF = 2048
E = 8
TOPK = 2
EPS = 1e-05

TB = 256
FB = 512
NEG = -1e30


def _qkv_kernel(h_ref, nw_ref, w_ref, qkv_ref):
    x = h_ref[...]
    nw = nw_ref[...]
    x = x * jax.lax.rsqrt(jnp.mean(x * x, axis=-1, keepdims=True) + EPS) * nw
    qkv_ref[...] = jax.lax.dot_general(
        x, w_ref[...], (((1,), (1,)), ((), ())),
        preferred_element_type=jnp.float32)


def _qkv_call(hidden, in_norm_w2d, qkv_w):
    return pl.pallas_call(
        _qkv_kernel,
        grid=(T // TB,),
        in_specs=[
            pl.BlockSpec((TB, DM), lambda i: (i, 0)),
            pl.BlockSpec((1, DM), lambda i: (0, 0)),
            pl.BlockSpec(((H + 2 * KVH) * D, DM), lambda i: (0, 0)),
        ],
        out_specs=pl.BlockSpec((TB, (H + 2 * KVH) * D), lambda i: (i, 0)),
        out_shape=jax.ShapeDtypeStruct((T, (H + 2 * KVH) * D), jnp.float32),
    )(hidden, in_norm_w2d, qkv_w)


def _rope(x, c, s):
    half = D // 2
    x1 = x[:, :half]
    x2 = x[:, half:]
    return jnp.concatenate([x1 * c - x2 * s, x2 * c + x1 * s], axis=-1)


def _attn_kernel(q_ref, k_ref, v_ref, cq_ref, sq_ref, ck_ref, sk_ref, o_ref):
    qb = pl.program_id(1)
    q = _rope(q_ref[0], cq_ref[...], sq_ref[...]) * (D ** -0.5)
    k = _rope(k_ref[0], ck_ref[...], sk_ref[...])
    s = jax.lax.dot_general(q, k, (((1,), (1,)), ((), ())),
                            preferred_element_type=jnp.float32)
    row = qb * TB + jax.lax.broadcasted_iota(jnp.int32, (TB, T), 0)
    col = jax.lax.broadcasted_iota(jnp.int32, (TB, T), 1)
    s = jnp.where(col <= row, s, NEG)
    m = jnp.max(s, axis=-1, keepdims=True)
    p = jnp.exp(s - m)
    p = p / jnp.sum(p, axis=-1, keepdims=True)
    o_ref[0] = jnp.dot(p, v_ref[0], preferred_element_type=jnp.float32)


def _attn_call(q3, k3, v3, cos, sin):
    return pl.pallas_call(
        _attn_kernel,
        grid=(H, T // TB),
        in_specs=[
            pl.BlockSpec((1, TB, D), lambda h, i: (h, i, 0)),
            pl.BlockSpec((1, T, D), lambda h, i: (h // (H // KVH), 0, 0)),
            pl.BlockSpec((1, T, D), lambda h, i: (h // (H // KVH), 0, 0)),
            pl.BlockSpec((TB, D // 2), lambda h, i: (i, 0)),
            pl.BlockSpec((TB, D // 2), lambda h, i: (i, 0)),
            pl.BlockSpec((T, D // 2), lambda h, i: (0, 0)),
            pl.BlockSpec((T, D // 2), lambda h, i: (0, 0)),
        ],
        out_specs=pl.BlockSpec((1, TB, D), lambda h, i: (h, i, 0)),
        out_shape=jax.ShapeDtypeStruct((H, T, D), jnp.float32),
    )(q3, k3, v3, cos, sin, cos, sin)


def _post_kernel(ctx_ref, res_ref, ow_ref, nw_ref, gw_ref,
                 ares_ref, x2_ref, topi_ref, topv_ref):
    attn = jax.lax.dot_general(ctx_ref[...], ow_ref[...], (((1,), (1,)), ((), ())),
                               preferred_element_type=jnp.float32)
    ares = attn + res_ref[...]
    ares_ref[...] = ares
    x2 = ares * jax.lax.rsqrt(jnp.mean(ares * ares, axis=-1, keepdims=True) + EPS)
    x2 = x2 * nw_ref[...]
    x2_ref[...] = x2
    logits = jax.lax.dot_general(x2, gw_ref[...], (((1,), (1,)), ((), ())),
                                 preferred_element_type=jnp.float32)
    m = jnp.max(logits, axis=-1, keepdims=True)
    p = jnp.exp(logits - m)
    probs = p / jnp.sum(p, axis=-1, keepdims=True)
    e_iota = jax.lax.broadcasted_iota(jnp.int32, (TB, E), 1)
    i1 = jnp.argmax(probs, axis=-1, keepdims=True)
    p1 = jnp.max(probs, axis=-1, keepdims=True)
    probs2 = jnp.where(e_iota == i1, -1.0, probs)
    i2 = jnp.argmax(probs2, axis=-1, keepdims=True)
    p2 = jnp.max(probs2, axis=-1, keepdims=True)
    tot = p1 + p2
    topi_ref[...] = jnp.concatenate([i1, i2], axis=-1)
    topv_ref[...] = jnp.concatenate([p1 / tot, p2 / tot], axis=-1)


def _post_call(ctx2d, hidden, o_w, post_norm_w2d, gate_w):
    return pl.pallas_call(
        _post_kernel,
        grid=(T // TB,),
        in_specs=[
            pl.BlockSpec((TB, H * D), lambda i: (i, 0)),
            pl.BlockSpec((TB, DM), lambda i: (i, 0)),
            pl.BlockSpec((DM, H * D), lambda i: (0, 0)),
            pl.BlockSpec((1, DM), lambda i: (0, 0)),
            pl.BlockSpec((E, DM), lambda i: (0, 0)),
        ],
        out_specs=[
            pl.BlockSpec((TB, DM), lambda i: (i, 0)),
            pl.BlockSpec((TB, DM), lambda i: (i, 0)),
            pl.BlockSpec((TB, TOPK), lambda i: (i, 0)),
            pl.BlockSpec((TB, TOPK), lambda i: (i, 0)),
        ],
        out_shape=[
            jax.ShapeDtypeStruct((T, DM), jnp.float32),
            jax.ShapeDtypeStruct((T, DM), jnp.float32),
            jax.ShapeDtypeStruct((T, TOPK), jnp.int32),
            jax.ShapeDtypeStruct((T, TOPK), jnp.float32),
        ],
    )(ctx2d, hidden, o_w, post_norm_w2d, gate_w)


RB = 256
NB = (T * TOPK + E * (RB - 1)) // RB
NF = DFF // FB


def _moe_kernel(be_ref, idc_ref, idr_ref, gp_ref, x2_ref,
                w1_ref, w3_ref, w2_ref, out_ref, xs_ref, acc_ref):
    b = pl.program_id(0)
    f = pl.program_id(1)

    @pl.when(jnp.logical_and(b == 0, f == 0))
    def _():
        out_ref[...] = jnp.zeros_like(out_ref)

    @pl.when(f == 0)
    def _():
        ids_col = idc_ref[0]
        t_iota = jax.lax.broadcasted_iota(jnp.int32, (RB, T), 1)
        oh = (t_iota == ids_col).astype(jnp.bfloat16)
        xs_ref[...] = jax.lax.dot_general(
            oh, x2_ref[...], (((1,), (0,)), ((), ())),
            preferred_element_type=jnp.float32).astype(jnp.bfloat16)
        acc_ref[...] = jnp.zeros_like(acc_ref)

    xs = xs_ref[...]
    g = jax.lax.dot_general(xs, w1_ref[0], (((1,), (1,)), ((), ())),
                            preferred_element_type=jnp.float32)
    u = jax.lax.dot_general(xs, w3_ref[0], (((1,), (1,)), ((), ())),
                            preferred_element_type=jnp.float32)
    act = ((g * jax.lax.logistic(g)) * u).astype(jnp.bfloat16)
    acc_ref[...] += jax.lax.dot_general(
        act, w2_ref[0], (((1,), (1,)), ((), ())),
        preferred_element_type=jnp.float32)

    @pl.when(f == NF - 1)
    def _():
        ids_row = idr_ref[0]
        t_iota = jax.lax.broadcasted_iota(jnp.int32, (T, RB), 0)
        oht = (t_iota == ids_row).astype(jnp.bfloat16)
        z = (acc_ref[...] * gp_ref[0]).astype(jnp.bfloat16)
        out_ref[...] += jax.lax.dot_general(
            oht, z, (((1,), (0,)), ((), ())),
            preferred_element_type=jnp.float32)


def _moe_call(block_expert, ids_col, ids_row, gates_p, x2_bf16, w1b, w3b, w2b):
    grid_spec = pltpu.PrefetchScalarGridSpec(
        num_scalar_prefetch=1,
        grid=(NB, NF),
        in_specs=[
            pl.BlockSpec((1, RB, 1), lambda b, f, be: (b, 0, 0)),
            pl.BlockSpec((1, 1, RB), lambda b, f, be: (b, 0, 0)),
            pl.BlockSpec((1, RB, 1), lambda b, f, be: (b, 0, 0)),
            pl.BlockSpec((T, DM), lambda b, f, be: (0, 0)),
            pl.BlockSpec((1, FB, DM), lambda b, f, be: (be[b], f, 0)),
            pl.BlockSpec((1, FB, DM), lambda b, f, be: (be[b], f, 0)),
            pl.BlockSpec((1, DM, FB), lambda b, f, be: (be[b], 0, f)),
        ],
        out_specs=pl.BlockSpec((T, DM), lambda b, f, be: (0, 0)),
        scratch_shapes=[
            pltpu.VMEM((RB, DM), jnp.bfloat16),
            pltpu.VMEM((RB, DM), jnp.float32),
        ],
    )
    return pl.pallas_call(
        _moe_kernel,
        grid_spec=grid_spec,
        out_shape=jax.ShapeDtypeStruct((T, DM), jnp.float32),
    )(block_expert, ids_col, ids_row, gates_p, x2_bf16, w1b, w3b, w2b)


def _routing_metadata(topi, topv):
    P = NB * RB
    flat_e = topi.reshape(-1)
    order = jnp.argsort(flat_e, stable=True)
    s_e = flat_e[order]
    tok = (order // TOPK).astype(jnp.int32)
    gval = jnp.take(topv.reshape(-1), order)
    counts = jnp.bincount(flat_e, length=E)
    padded = ((counts + RB - 1) // RB) * RB
    cum_p = jnp.cumsum(padded)
    offs = cum_p - padded
    cum_c = jnp.cumsum(counts) - counts
    p_idx = jnp.arange(P, dtype=jnp.int32)
    eb_p = jnp.minimum(
        jnp.searchsorted(cum_p, p_idx, side="right"), E - 1).astype(jnp.int32)
    within = p_idx - offs[eb_p]
    valid = within < counts[eb_p]
    j = jnp.clip(cum_c[eb_p] + within, 0, T * TOPK - 1)
    row_ids = jnp.where(valid, jnp.take(tok, j), T)
    gates_p = jnp.where(valid, jnp.take(gval, j), 0.0)
    block_expert = eb_p[::RB]
    return (block_expert,
            row_ids.reshape(NB, RB, 1),
            row_ids.reshape(NB, 1, RB),
            gates_p.reshape(NB, RB, 1).astype(jnp.float32))


@jax.jit
def kernel(hidden_states, cos, sin, in_norm_w, post_norm_w, qkv_w, o_w,
           gate_w, w1, w3, w2):
    qkv = _qkv_call(hidden_states, in_norm_w.reshape(1, DM), qkv_w)
    q3 = qkv[:, :H * D].reshape(T, H, D).transpose(1, 0, 2)
    k3 = qkv[:, H * D:(H + KVH) * D].reshape(T, KVH, D).transpose(1, 0, 2)
    v3 = qkv[:, (H + KVH) * D:].reshape(T, KVH, D).transpose(1, 0, 2)
    ctx = _attn_call(q3, k3, v3, cos, sin)
    ctx2d = ctx.transpose(1, 0, 2).reshape(T, H * D)
    attn_res, x2, topi, topv = _post_call(ctx2d, hidden_states, o_w,
                                          post_norm_w.reshape(1, DM), gate_w)
    block_expert, ids_col, ids_row, gates_p = _routing_metadata(topi, topv)
    moe_out = _moe_call(block_expert, ids_col, ids_row, gates_p,
                        x2.astype(jnp.bfloat16),
                        w1.astype(jnp.bfloat16), w3.astype(jnp.bfloat16),
                        w2.astype(jnp.bfloat16))
    return (moe_out, attn_res)


# speedup vs baseline: 1.5623x; 1.3597x over previous
import functools

import jax
import jax.numpy as jnp
from jax.experimental import pallas as pl
from jax.experimental.pallas import tpu as pltpu

T = 2048
DM = 1024
H = 16
KVH = 8
D = 64
DFF = 2048
E = 8
TOPK = 2
EPS = 1e-05

TB = 256
FB = 2048
NEG = -1e30


def _qkv_kernel(h_ref, nw_ref, w_ref, qkv_ref):
    x = h_ref[...]
    nw = nw_ref[...]
    x = x * jax.lax.rsqrt(jnp.mean(x * x, axis=-1, keepdims=True) + EPS) * nw
    qkv_ref[...] = jax.lax.dot_general(
        x, w_ref[...], (((1,), (1,)), ((), ())),
        preferred_element_type=jnp.float32)


def _qkv_call(hidden, in_norm_w2d, qkv_w):
    return pl.pallas_call(
        _qkv_kernel,
        grid=(T // TB,),
        in_specs=[
            pl.BlockSpec((TB, DM), lambda i: (i, 0)),
            pl.BlockSpec((1, DM), lambda i: (0, 0)),
            pl.BlockSpec(((H + 2 * KVH) * D, DM), lambda i: (0, 0)),
        ],
        out_specs=pl.BlockSpec((TB, (H + 2 * KVH) * D), lambda i: (i, 0)),
        out_shape=jax.ShapeDtypeStruct((T, (H + 2 * KVH) * D), jnp.float32),
    )(hidden, in_norm_w2d, qkv_w)


GH = 4
NG = H // GH
KB = TB


def _rope_heads(x, c, s, nheads):
    parts = []
    for j in range(nheads):
        x1 = x[:, j * D:j * D + D // 2]
        x2 = x[:, j * D + D // 2:(j + 1) * D]
        parts += [x1 * c - x2 * s, x2 * c + x1 * s]
    return jnp.concatenate(parts, axis=1)


def _attn_kernel(q_ref, k_ref, v_ref, cq_ref, sq_ref, ck_ref, sk_ref, o_ref):
    qb = pl.program_id(1)
    q = _rope_heads(q_ref[...], cq_ref[...], sq_ref[...], GH)
    k = _rope_heads(k_ref[...], ck_ref[...], sk_ref[...], GH // 2)
    v = v_ref[...]
    row = qb * TB + jax.lax.broadcasted_iota(jnp.int32, (TB, T), 0)
    col = jax.lax.broadcasted_iota(jnp.int32, (TB, T), 1)
    ok = col <= row
    for h in range(GH):
        kv = h // 2
        qh = q[:, h * D:(h + 1) * D]
        kh = k[:, kv * D:(kv + 1) * D]
        vh = v[:, kv * D:(kv + 1) * D]
        sc = jax.lax.dot_general(qh, kh, (((1,), (1,)), ((), ())),
                                 preferred_element_type=jnp.float32)
        sc = jnp.where(ok, sc * (D ** -0.5), NEG)
        m = jnp.max(sc, axis=1, keepdims=True)
        p = jnp.exp(sc - m)
        p = p / jnp.sum(p, axis=1, keepdims=True)
        o_ref[:, h * D:(h + 1) * D] = jnp.dot(
            p, vh, preferred_element_type=jnp.float32)


def _attn_call(qkv, cos, sin):
    return pl.pallas_call(
        _attn_kernel,
        grid=(NG, T // TB),
        in_specs=[
            pl.BlockSpec((TB, GH * D), lambda g, i: (i, g)),
            pl.BlockSpec((T, GH // 2 * D), lambda g, i: (0, (H + g * 2) // 2)),
            pl.BlockSpec((T, GH // 2 * D), lambda g, i: (0, (H + KVH + g * 2) // 2)),
            pl.BlockSpec((TB, D // 2), lambda g, i: (i, 0)),
            pl.BlockSpec((TB, D // 2), lambda g, i: (i, 0)),
            pl.BlockSpec((T, D // 2), lambda g, i: (0, 0)),
            pl.BlockSpec((T, D // 2), lambda g, i: (0, 0)),
        ],
        out_specs=pl.BlockSpec((TB, GH * D), lambda g, i: (i, g)),
        out_shape=jax.ShapeDtypeStruct((T, H * D), jnp.float32),
    )(qkv, qkv, qkv, cos, sin, cos, sin)


def _post_kernel(ctx_ref, res_ref, ow_ref, nw_ref, gw_ref,
                 ares_ref, x2_ref, topi_ref, topv_ref):
    attn = jax.lax.dot_general(ctx_ref[...], ow_ref[...], (((1,), (1,)), ((), ())),
                               preferred_element_type=jnp.float32)
    ares = attn + res_ref[...]
    ares_ref[...] = ares
    x2 = ares * jax.lax.rsqrt(jnp.mean(ares * ares, axis=-1, keepdims=True) + EPS)
    x2 = x2 * nw_ref[...]
    x2_ref[...] = x2
    logits = jax.lax.dot_general(x2, gw_ref[...], (((1,), (1,)), ((), ())),
                                 preferred_element_type=jnp.float32)
    m = jnp.max(logits, axis=-1, keepdims=True)
    p = jnp.exp(logits - m)
    probs = p / jnp.sum(p, axis=-1, keepdims=True)
    e_iota = jax.lax.broadcasted_iota(jnp.int32, (TB, E), 1)
    i1 = jnp.argmax(probs, axis=-1, keepdims=True)
    p1 = jnp.max(probs, axis=-1, keepdims=True)
    probs2 = jnp.where(e_iota == i1, -1.0, probs)
    i2 = jnp.argmax(probs2, axis=-1, keepdims=True)
    p2 = jnp.max(probs2, axis=-1, keepdims=True)
    tot = p1 + p2
    topi_ref[...] = jnp.concatenate([i1, i2], axis=-1)
    topv_ref[...] = jnp.concatenate([p1 / tot, p2 / tot], axis=-1)


def _post_call(ctx2d, hidden, o_w, post_norm_w2d, gate_w):
    return pl.pallas_call(
        _post_kernel,
        grid=(T // TB,),
        in_specs=[
            pl.BlockSpec((TB, H * D), lambda i: (i, 0)),
            pl.BlockSpec((TB, DM), lambda i: (i, 0)),
            pl.BlockSpec((DM, H * D), lambda i: (0, 0)),
            pl.BlockSpec((1, DM), lambda i: (0, 0)),
            pl.BlockSpec((E, DM), lambda i: (0, 0)),
        ],
        out_specs=[
            pl.BlockSpec((TB, DM), lambda i: (i, 0)),
            pl.BlockSpec((TB, DM), lambda i: (i, 0)),
            pl.BlockSpec((TB, TOPK), lambda i: (i, 0)),
            pl.BlockSpec((TB, TOPK), lambda i: (i, 0)),
        ],
        out_shape=[
            jax.ShapeDtypeStruct((T, DM), jnp.float32),
            jax.ShapeDtypeStruct((T, DM), jnp.float32),
            jax.ShapeDtypeStruct((T, TOPK), jnp.int32),
            jax.ShapeDtypeStruct((T, TOPK), jnp.float32),
        ],
    )(ctx2d, hidden, o_w, post_norm_w2d, gate_w)


RB = 256
NB = (T * TOPK + E * (RB - 1)) // RB
NF = DFF // FB


def _moe_kernel(be_ref, idc_ref, idr_ref, gp_ref, x2_ref,
                w1_ref, w3_ref, w2_ref, out_ref, xs_ref, acc_ref):
    b = pl.program_id(0)
    f = pl.program_id(1)

    @pl.when(jnp.logical_and(b == 0, f == 0))
    def _():
        out_ref[...] = jnp.zeros_like(out_ref)

    @pl.when(f == 0)
    def _():
        ids_col = idc_ref[0]
        t_iota = jax.lax.broadcasted_iota(jnp.int32, (RB, T), 1)
        oh = (t_iota == ids_col).astype(jnp.bfloat16)
        xs_ref[...] = jax.lax.dot_general(
            oh, x2_ref[...], (((1,), (0,)), ((), ())),
            preferred_element_type=jnp.float32).astype(jnp.bfloat16)
        acc_ref[...] = jnp.zeros_like(acc_ref)

    xs = xs_ref[...]
    g = jax.lax.dot_general(xs, w1_ref[0], (((1,), (1,)), ((), ())),
                            preferred_element_type=jnp.float32)
    u = jax.lax.dot_general(xs, w3_ref[0], (((1,), (1,)), ((), ())),
                            preferred_element_type=jnp.float32)
    act = ((g * jax.lax.logistic(g)) * u).astype(jnp.bfloat16)
    acc_ref[...] += jax.lax.dot_general(
        act, w2_ref[0], (((1,), (1,)), ((), ())),
        preferred_element_type=jnp.float32)

    @pl.when(f == NF - 1)
    def _():
        ids_row = idr_ref[0]
        t_iota = jax.lax.broadcasted_iota(jnp.int32, (T, RB), 0)
        oht = (t_iota == ids_row).astype(jnp.bfloat16)
        z = (acc_ref[...] * gp_ref[0]).astype(jnp.bfloat16)
        out_ref[...] += jax.lax.dot_general(
            oht, z, (((1,), (0,)), ((), ())),
            preferred_element_type=jnp.float32)


def _moe_call(block_expert, ids_col, ids_row, gates_p, x2_bf16, w1b, w3b, w2b):
    grid_spec = pltpu.PrefetchScalarGridSpec(
        num_scalar_prefetch=1,
        grid=(NB, NF),
        in_specs=[
            pl.BlockSpec((1, RB, 1), lambda b, f, be: (b, 0, 0)),
            pl.BlockSpec((1, 1, RB), lambda b, f, be: (b, 0, 0)),
            pl.BlockSpec((1, RB, 1), lambda b, f, be: (b, 0, 0)),
            pl.BlockSpec((T, DM), lambda b, f, be: (0, 0)),
            pl.BlockSpec((1, FB, DM), lambda b, f, be: (be[b], f, 0)),
            pl.BlockSpec((1, FB, DM), lambda b, f, be: (be[b], f, 0)),
            pl.BlockSpec((1, DM, FB), lambda b, f, be: (be[b], 0, f)),
        ],
        out_specs=pl.BlockSpec((T, DM), lambda b, f, be: (0, 0)),
        scratch_shapes=[
            pltpu.VMEM((RB, DM), jnp.bfloat16),
            pltpu.VMEM((RB, DM), jnp.float32),
        ],
    )
    return pl.pallas_call(
        _moe_kernel,
        grid_spec=grid_spec,
        out_shape=jax.ShapeDtypeStruct((T, DM), jnp.float32),
    )(block_expert, ids_col, ids_row, gates_p, x2_bf16, w1b, w3b, w2b)


def _routing_metadata(topi, topv):
    P = NB * RB
    flat_e = topi.reshape(-1)
    order = jnp.argsort(flat_e, stable=True)
    s_e = flat_e[order]
    tok = (order // TOPK).astype(jnp.int32)
    gval = jnp.take(topv.reshape(-1), order)
    counts = jnp.bincount(flat_e, length=E)
    padded = ((counts + RB - 1) // RB) * RB
    cum_p = jnp.cumsum(padded)
    offs = cum_p - padded
    cum_c = jnp.cumsum(counts) - counts
    p_idx = jnp.arange(P, dtype=jnp.int32)
    eb_p = jnp.minimum(
        jnp.searchsorted(cum_p, p_idx, side="right"), E - 1).astype(jnp.int32)
    within = p_idx - offs[eb_p]
    valid = within < counts[eb_p]
    j = jnp.clip(cum_c[eb_p] + within, 0, T * TOPK - 1)
    row_ids = jnp.where(valid, jnp.take(tok, j), T)
    gates_p = jnp.where(valid, jnp.take(gval, j), 0.0)
    block_expert = eb_p[::RB]
    return (block_expert,
            row_ids.reshape(NB, RB, 1),
            row_ids.reshape(NB, 1, RB),
            gates_p.reshape(NB, RB, 1).astype(jnp.float32))


@jax.jit
def kernel(hidden_states, cos, sin, in_norm_w, post_norm_w, qkv_w, o_w,
           gate_w, w1, w3, w2):
    qkv = _qkv_call(hidden_states, in_norm_w.reshape(1, DM), qkv_w)
    ctx2d = _attn_call(qkv, cos, sin)
    attn_res, x2, topi, topv = _post_call(ctx2d, hidden_states, o_w,
                                          post_norm_w.reshape(1, DM), gate_w)
    block_expert, ids_col, ids_row, gates_p = _routing_metadata(topi, topv)
    moe_out = _moe_call(block_expert, ids_col, ids_row, gates_p,
                        x2.astype(jnp.bfloat16),
                        w1.astype(jnp.bfloat16), w3.astype(jnp.bfloat16),
                        w2.astype(jnp.bfloat16))
    return (moe_out, attn_res)


# speedup vs baseline: 1.9740x; 1.2635x over previous
import functools

import jax
import jax.numpy as jnp
from jax.experimental import pallas as pl
from jax.experimental.pallas import tpu as pltpu

T = 2048
DM = 1024
H = 16
KVH = 8
D = 64
DFF = 2048
E = 8
TOPK = 2
EPS = 1e-05

TB = 256
FB = 1024
NEG = -1e30


def _qkv_kernel(h_ref, nw_ref, w_ref, qkv_ref):
    x = h_ref[...]
    nw = nw_ref[...]
    x = x * jax.lax.rsqrt(jnp.mean(x * x, axis=-1, keepdims=True) + EPS) * nw
    qkv_ref[...] = jax.lax.dot_general(
        x, w_ref[...], (((1,), (1,)), ((), ())),
        preferred_element_type=jnp.float32)


def _qkv_call(hidden, in_norm_w2d, qkv_w):
    return pl.pallas_call(
        _qkv_kernel,
        grid=(T // TB,),
        in_specs=[
            pl.BlockSpec((TB, DM), lambda i: (i, 0)),
            pl.BlockSpec((1, DM), lambda i: (0, 0)),
            pl.BlockSpec(((H + 2 * KVH) * D, DM), lambda i: (0, 0)),
        ],
        out_specs=pl.BlockSpec((TB, (H + 2 * KVH) * D), lambda i: (i, 0)),
        out_shape=jax.ShapeDtypeStruct((T, (H + 2 * KVH) * D), jnp.float32),
    )(hidden, in_norm_w2d, qkv_w)


GH = 4
NG = H // GH
ATB = 512


def _rope_heads(x, c, s, nheads):
    parts = []
    for j in range(nheads):
        x1 = x[:, j * D:j * D + D // 2]
        x2 = x[:, j * D + D // 2:(j + 1) * D]
        parts += [x1 * c - x2 * s, x2 * c + x1 * s]
    return jnp.concatenate(parts, axis=1)


def _attn_kernel(q_ref, k_ref, v_ref, cq_ref, sq_ref, ck_ref, sk_ref, o_ref):
    qb = pl.program_id(1)
    q = _rope_heads(q_ref[...], cq_ref[...], sq_ref[...], GH)
    k = _rope_heads(k_ref[...], ck_ref[...], sk_ref[...], GH // 2)
    v = v_ref[...]
    row = qb * ATB + jax.lax.broadcasted_iota(jnp.int32, (ATB, T), 0)
    col = jax.lax.broadcasted_iota(jnp.int32, (ATB, T), 1)
    ok = col <= row
    for h in range(GH):
        kv = h // 2
        qh = q[:, h * D:(h + 1) * D]
        kh = k[:, kv * D:(kv + 1) * D]
        vh = v[:, kv * D:(kv + 1) * D]
        sc = jax.lax.dot_general(qh, kh, (((1,), (1,)), ((), ())),
                                 preferred_element_type=jnp.float32)
        sc = jnp.where(ok, sc * (D ** -0.5), NEG)
        m = jnp.max(sc, axis=1, keepdims=True)
        p = jnp.exp(sc - m)
        p = p / jnp.sum(p, axis=1, keepdims=True)
        o_ref[:, h * D:(h + 1) * D] = jnp.dot(
            p, vh, preferred_element_type=jnp.float32)


def _attn_call(qkv, cos, sin):
    return pl.pallas_call(
        _attn_kernel,
        grid=(NG, T // ATB),
        in_specs=[
            pl.BlockSpec((ATB, GH * D), lambda g, i: (i, g)),
            pl.BlockSpec((T, GH // 2 * D), lambda g, i: (0, (H + g * 2) // 2)),
            pl.BlockSpec((T, GH // 2 * D), lambda g, i: (0, (H + KVH + g * 2) // 2)),
            pl.BlockSpec((ATB, D // 2), lambda g, i: (i, 0)),
            pl.BlockSpec((ATB, D // 2), lambda g, i: (i, 0)),
            pl.BlockSpec((T, D // 2), lambda g, i: (0, 0)),
            pl.BlockSpec((T, D // 2), lambda g, i: (0, 0)),
        ],
        out_specs=pl.BlockSpec((ATB, GH * D), lambda g, i: (i, g)),
        out_shape=jax.ShapeDtypeStruct((T, H * D), jnp.float32),
    )(qkv, qkv, qkv, cos, sin, cos, sin)


def _post_kernel(ctx_ref, res_ref, ow_ref, nw_ref, gw_ref,
                 ares_ref, x2_ref, topi_ref, topv_ref):
    attn = jax.lax.dot_general(ctx_ref[...], ow_ref[...], (((1,), (1,)), ((), ())),
                               preferred_element_type=jnp.float32)
    ares = attn + res_ref[...]
    ares_ref[...] = ares
    x2 = ares * jax.lax.rsqrt(jnp.mean(ares * ares, axis=-1, keepdims=True) + EPS)
    x2 = x2 * nw_ref[...]
    x2_ref[...] = x2
    logits = jax.lax.dot_general(x2, gw_ref[...], (((1,), (1,)), ((), ())),
                                 preferred_element_type=jnp.float32)
    m = jnp.max(logits, axis=-1, keepdims=True)
    p = jnp.exp(logits - m)
    probs = p / jnp.sum(p, axis=-1, keepdims=True)
    e_iota = jax.lax.broadcasted_iota(jnp.int32, (TB, E), 1)
    i1 = jnp.argmax(probs, axis=-1, keepdims=True)
    p1 = jnp.max(probs, axis=-1, keepdims=True)
    probs2 = jnp.where(e_iota == i1, -1.0, probs)
    i2 = jnp.argmax(probs2, axis=-1, keepdims=True)
    p2 = jnp.max(probs2, axis=-1, keepdims=True)
    tot = p1 + p2
    topi_ref[...] = jnp.concatenate([i1, i2], axis=-1)
    topv_ref[...] = jnp.concatenate([p1 / tot, p2 / tot], axis=-1)


def _post_call(ctx2d, hidden, o_w, post_norm_w2d, gate_w):
    return pl.pallas_call(
        _post_kernel,
        grid=(T // TB,),
        in_specs=[
            pl.BlockSpec((TB, H * D), lambda i: (i, 0)),
            pl.BlockSpec((TB, DM), lambda i: (i, 0)),
            pl.BlockSpec((DM, H * D), lambda i: (0, 0)),
            pl.BlockSpec((1, DM), lambda i: (0, 0)),
            pl.BlockSpec((E, DM), lambda i: (0, 0)),
        ],
        out_specs=[
            pl.BlockSpec((TB, DM), lambda i: (i, 0)),
            pl.BlockSpec((TB, DM), lambda i: (i, 0)),
            pl.BlockSpec((TB, TOPK), lambda i: (i, 0)),
            pl.BlockSpec((TB, TOPK), lambda i: (i, 0)),
        ],
        out_shape=[
            jax.ShapeDtypeStruct((T, DM), jnp.float32),
            jax.ShapeDtypeStruct((T, DM), jnp.float32),
            jax.ShapeDtypeStruct((T, TOPK), jnp.int32),
            jax.ShapeDtypeStruct((T, TOPK), jnp.float32),
        ],
    )(ctx2d, hidden, o_w, post_norm_w2d, gate_w)


RB = 256
NB = (T * TOPK + E * (RB - 1)) // RB
NF = DFF // FB


def _moe_kernel(be_ref, idc_ref, idr_ref, gp_ref, x2_ref,
                w1_ref, w3_ref, w2_ref, out_ref, xs_ref, acc_ref):
    b = pl.program_id(0)
    f = pl.program_id(1)

    @pl.when(jnp.logical_and(b == 0, f == 0))
    def _():
        out_ref[...] = jnp.zeros_like(out_ref)

    @pl.when(f == 0)
    def _():
        ids_col = idc_ref[0]
        t_iota = jax.lax.broadcasted_iota(jnp.int32, (RB, T), 1)
        oh = (t_iota == ids_col).astype(jnp.bfloat16)
        xs_ref[...] = jax.lax.dot_general(
            oh, x2_ref[...], (((1,), (0,)), ((), ())),
            preferred_element_type=jnp.float32).astype(jnp.bfloat16)
        acc_ref[...] = jnp.zeros_like(acc_ref)

    xs = xs_ref[...]
    g = jax.lax.dot_general(xs, w1_ref[0].astype(jnp.bfloat16),
                            (((1,), (1,)), ((), ())),
                            preferred_element_type=jnp.float32)
    u = jax.lax.dot_general(xs, w3_ref[0].astype(jnp.bfloat16),
                            (((1,), (1,)), ((), ())),
                            preferred_element_type=jnp.float32)
    act = ((g * jax.lax.logistic(g)) * u).astype(jnp.bfloat16)
    acc_ref[...] += jax.lax.dot_general(
        act, w2_ref[0].astype(jnp.bfloat16), (((1,), (1,)), ((), ())),
        preferred_element_type=jnp.float32)

    @pl.when(f == NF - 1)
    def _():
        ids_row = idr_ref[0]
        t_iota = jax.lax.broadcasted_iota(jnp.int32, (T, RB), 0)
        oht = (t_iota == ids_row).astype(jnp.bfloat16)
        z = (acc_ref[...] * gp_ref[0]).astype(jnp.bfloat16)
        out_ref[...] += jax.lax.dot_general(
            oht, z, (((1,), (0,)), ((), ())),
            preferred_element_type=jnp.float32)


def _moe_call(block_expert, ids_col, ids_row, gates_p, x2_bf16, w1b, w3b, w2b):
    grid_spec = pltpu.PrefetchScalarGridSpec(
        num_scalar_prefetch=1,
        grid=(NB, NF),
        in_specs=[
            pl.BlockSpec((1, RB, 1), lambda b, f, be: (b, 0, 0)),
            pl.BlockSpec((1, 1, RB), lambda b, f, be: (b, 0, 0)),
            pl.BlockSpec((1, RB, 1), lambda b, f, be: (b, 0, 0)),
            pl.BlockSpec((T, DM), lambda b, f, be: (0, 0)),
            pl.BlockSpec((1, FB, DM), lambda b, f, be: (be[b], f, 0)),
            pl.BlockSpec((1, FB, DM), lambda b, f, be: (be[b], f, 0)),
            pl.BlockSpec((1, DM, FB), lambda b, f, be: (be[b], 0, f)),
        ],
        out_specs=pl.BlockSpec((T, DM), lambda b, f, be: (0, 0)),
        scratch_shapes=[
            pltpu.VMEM((RB, DM), jnp.bfloat16),
            pltpu.VMEM((RB, DM), jnp.float32),
        ],
    )
    return pl.pallas_call(
        _moe_kernel,
        grid_spec=grid_spec,
        out_shape=jax.ShapeDtypeStruct((T, DM), jnp.float32),
    )(block_expert, ids_col, ids_row, gates_p, x2_bf16, w1b, w3b, w2b)


def _routing_metadata(topi, topv):
    P = NB * RB
    flat_e = topi.reshape(-1)
    order = jnp.argsort(flat_e, stable=True).astype(jnp.int32)
    e_iota = jnp.arange(E, dtype=jnp.int32)
    counts = jnp.sum(flat_e[:, None] == e_iota[None, :], axis=0,
                     dtype=jnp.int32)
    padded = ((counts + RB - 1) // RB) * RB
    cum_p = jnp.cumsum(padded)
    offs = cum_p - padded
    cum_c = jnp.cumsum(counts) - counts
    p_idx = jnp.arange(P, dtype=jnp.int32)
    eb_oh = (p_idx[:, None] >= cum_p[None, :])
    eb_p = jnp.minimum(jnp.sum(eb_oh, axis=1, dtype=jnp.int32), E - 1)
    sel = (eb_p[:, None] == e_iota[None, :])
    offs_p = jnp.sum(jnp.where(sel, offs[None, :], 0), axis=1)
    counts_p = jnp.sum(jnp.where(sel, counts[None, :], 0), axis=1)
    cum_c_p = jnp.sum(jnp.where(sel, cum_c[None, :], 0), axis=1)
    within = p_idx - offs_p
    valid = within < counts_p
    j = jnp.clip(cum_c_p + within, 0, T * TOPK - 1)
    oj = jnp.take(order, j)
    row_ids = jnp.where(valid, oj // TOPK, T)
    gates_p = jnp.where(valid, jnp.take(topv.reshape(-1), oj), 0.0)
    block_expert = eb_p[::RB]
    return (block_expert,
            row_ids.reshape(NB, RB, 1),
            row_ids.reshape(NB, 1, RB),
            gates_p.reshape(NB, RB, 1).astype(jnp.float32))


@jax.jit
def kernel(hidden_states, cos, sin, in_norm_w, post_norm_w, qkv_w, o_w,
           gate_w, w1, w3, w2):
    qkv = _qkv_call(hidden_states, in_norm_w.reshape(1, DM), qkv_w)
    ctx2d = _attn_call(qkv, cos, sin)
    attn_res, x2, topi, topv = _post_call(ctx2d, hidden_states, o_w,
                                          post_norm_w.reshape(1, DM), gate_w)
    block_expert, ids_col, ids_row, gates_p = _routing_metadata(topi, topv)
    moe_out = _moe_call(block_expert, ids_col, ids_row, gates_p,
                        x2.astype(jnp.bfloat16), w1, w3, w2)
    return (moe_out, attn_res)


# speedup vs baseline: 2.0733x; 1.0503x over previous
import functools

import jax
import jax.numpy as jnp
from jax.experimental import pallas as pl
from jax.experimental.pallas import tpu as pltpu

T = 2048
DM = 1024
H = 16
KVH = 8
D = 64
DFF = 2048
E = 8
TOPK = 2
EPS = 1e-05

TB = 256
FB = 1024
NEG = -1e30


def _qkv_kernel(h_ref, nw_ref, w_ref, qkv_ref):
    x = h_ref[...]
    nw = nw_ref[...]
    x = x * jax.lax.rsqrt(jnp.mean(x * x, axis=-1, keepdims=True) + EPS) * nw
    qkv_ref[...] = jax.lax.dot_general(
        x, w_ref[...], (((1,), (1,)), ((), ())),
        preferred_element_type=jnp.float32)


def _qkv_call(hidden, in_norm_w2d, qkv_w):
    return pl.pallas_call(
        _qkv_kernel,
        grid=(T // TB,),
        in_specs=[
            pl.BlockSpec((TB, DM), lambda i: (i, 0)),
            pl.BlockSpec((1, DM), lambda i: (0, 0)),
            pl.BlockSpec(((H + 2 * KVH) * D, DM), lambda i: (0, 0)),
        ],
        out_specs=pl.BlockSpec((TB, (H + 2 * KVH) * D), lambda i: (i, 0)),
        out_shape=jax.ShapeDtypeStruct((T, (H + 2 * KVH) * D), jnp.float32),
    )(hidden, in_norm_w2d, qkv_w)


GH = 4
NG = H // GH
ATB = 512


def _rope_heads(x, c, s, nheads):
    parts = []
    for j in range(nheads):
        x1 = x[:, j * D:j * D + D // 2]
        x2 = x[:, j * D + D // 2:(j + 1) * D]
        parts += [x1 * c - x2 * s, x2 * c + x1 * s]
    return jnp.concatenate(parts, axis=1)


def _attn_kernel(q_ref, k_ref, v_ref, cq_ref, sq_ref, ck_ref, sk_ref, o_ref,
                 *, row_off, klen):
    qb = pl.program_id(1)
    q = _rope_heads(q_ref[...], cq_ref[...], sq_ref[...], GH)
    k = _rope_heads(k_ref[...], ck_ref[...], sk_ref[...], GH // 2)
    v = v_ref[...]
    row = row_off + qb * ATB + jax.lax.broadcasted_iota(
        jnp.int32, (ATB, klen), 0)
    col = jax.lax.broadcasted_iota(jnp.int32, (ATB, klen), 1)
    ok = col <= row
    for h in range(GH):
        kv = h // 2
        qh = q[:, h * D:(h + 1) * D]
        kh = k[:, kv * D:(kv + 1) * D]
        vh = v[:, kv * D:(kv + 1) * D]
        sc = jax.lax.dot_general(qh, kh, (((1,), (1,)), ((), ())),
                                 preferred_element_type=jnp.float32)
        sc = jnp.where(ok, sc * (D ** -0.5), NEG)
        m = jnp.max(sc, axis=1, keepdims=True)
        p = jnp.exp(sc - m)
        p = p / jnp.sum(p, axis=1, keepdims=True)
        o_ref[:, h * D:(h + 1) * D] = jnp.dot(
            p, vh, preferred_element_type=jnp.float32)


def _attn_call_part(qkv, cos, sin, row_off, nrows, klen):
    ob = row_off // ATB
    return pl.pallas_call(
        functools.partial(_attn_kernel, row_off=row_off, klen=klen),
        grid=(NG, nrows // ATB),
        in_specs=[
            pl.BlockSpec((ATB, GH * D), lambda g, i: (ob + i, g)),
            pl.BlockSpec((klen, GH // 2 * D), lambda g, i: (0, (H + g * 2) // 2)),
            pl.BlockSpec((klen, GH // 2 * D), lambda g, i: (0, (H + KVH + g * 2) // 2)),
            pl.BlockSpec((ATB, D // 2), lambda g, i: (ob + i, 0)),
            pl.BlockSpec((ATB, D // 2), lambda g, i: (ob + i, 0)),
            pl.BlockSpec((klen, D // 2), lambda g, i: (0, 0)),
            pl.BlockSpec((klen, D // 2), lambda g, i: (0, 0)),
        ],
        out_specs=pl.BlockSpec((ATB, GH * D), lambda g, i: (i, g)),
        out_shape=jax.ShapeDtypeStruct((nrows, H * D), jnp.float32),
    )(qkv, qkv, qkv, cos, sin, cos, sin)


def _attn_call(qkv, cos, sin):
    lo = _attn_call_part(qkv, cos, sin, 0, T // 2, T // 2)
    hi = _attn_call_part(qkv, cos, sin, T // 2, T // 2, T)
    return jnp.concatenate([lo, hi], axis=0)


def _post_kernel(ctx_ref, res_ref, ow_ref, nw_ref, gw_ref,
                 ares_ref, x2_ref, topi_ref, topv_ref):
    attn = jax.lax.dot_general(ctx_ref[...], ow_ref[...], (((1,), (1,)), ((), ())),
                               preferred_element_type=jnp.float32)
    ares = attn + res_ref[...]
    ares_ref[...] = ares
    x2 = ares * jax.lax.rsqrt(jnp.mean(ares * ares, axis=-1, keepdims=True) + EPS)
    x2 = x2 * nw_ref[...]
    x2_ref[...] = x2
    logits = jax.lax.dot_general(x2, gw_ref[...], (((1,), (1,)), ((), ())),
                                 preferred_element_type=jnp.float32)
    m = jnp.max(logits, axis=-1, keepdims=True)
    p = jnp.exp(logits - m)
    probs = p / jnp.sum(p, axis=-1, keepdims=True)
    e_iota = jax.lax.broadcasted_iota(jnp.int32, (TB, E), 1)
    i1 = jnp.argmax(probs, axis=-1, keepdims=True)
    p1 = jnp.max(probs, axis=-1, keepdims=True)
    probs2 = jnp.where(e_iota == i1, -1.0, probs)
    i2 = jnp.argmax(probs2, axis=-1, keepdims=True)
    p2 = jnp.max(probs2, axis=-1, keepdims=True)
    tot = p1 + p2
    topi_ref[...] = jnp.concatenate([i1, i2], axis=-1)
    topv_ref[...] = jnp.concatenate([p1 / tot, p2 / tot], axis=-1)


def _post_call(ctx2d, hidden, o_w, post_norm_w2d, gate_w):
    return pl.pallas_call(
        _post_kernel,
        grid=(T // TB,),
        in_specs=[
            pl.BlockSpec((TB, H * D), lambda i: (i, 0)),
            pl.BlockSpec((TB, DM), lambda i: (i, 0)),
            pl.BlockSpec((DM, H * D), lambda i: (0, 0)),
            pl.BlockSpec((1, DM), lambda i: (0, 0)),
            pl.BlockSpec((E, DM), lambda i: (0, 0)),
        ],
        out_specs=[
            pl.BlockSpec((TB, DM), lambda i: (i, 0)),
            pl.BlockSpec((TB, DM), lambda i: (i, 0)),
            pl.BlockSpec((TB, TOPK), lambda i: (i, 0)),
            pl.BlockSpec((TB, TOPK), lambda i: (i, 0)),
        ],
        out_shape=[
            jax.ShapeDtypeStruct((T, DM), jnp.float32),
            jax.ShapeDtypeStruct((T, DM), jnp.float32),
            jax.ShapeDtypeStruct((T, TOPK), jnp.int32),
            jax.ShapeDtypeStruct((T, TOPK), jnp.float32),
        ],
    )(ctx2d, hidden, o_w, post_norm_w2d, gate_w)


RB = 256
NB = (T * TOPK + E * (RB - 1)) // RB
NF = DFF // FB


def _moe_kernel(be_ref, idc_ref, idr_ref, gp_ref, x2_ref,
                w1_ref, w3_ref, w2_ref, out_ref, xs_ref, acc_ref):
    b = pl.program_id(0)
    f = pl.program_id(1)

    @pl.when(jnp.logical_and(b == 0, f == 0))
    def _():
        out_ref[...] = jnp.zeros_like(out_ref)

    @pl.when(f == 0)
    def _():
        ids_col = idc_ref[0]
        t_iota = jax.lax.broadcasted_iota(jnp.int32, (RB, T), 1)
        oh = (t_iota == ids_col).astype(jnp.bfloat16)
        xs_ref[...] = jax.lax.dot_general(
            oh, x2_ref[...], (((1,), (0,)), ((), ())),
            preferred_element_type=jnp.float32).astype(jnp.bfloat16)
        acc_ref[...] = jnp.zeros_like(acc_ref)

    xs = xs_ref[...]
    g = jax.lax.dot_general(xs, w1_ref[0].astype(jnp.bfloat16),
                            (((1,), (1,)), ((), ())),
                            preferred_element_type=jnp.float32)
    u = jax.lax.dot_general(xs, w3_ref[0].astype(jnp.bfloat16),
                            (((1,), (1,)), ((), ())),
                            preferred_element_type=jnp.float32)
    act = ((g * jax.lax.logistic(g)) * u).astype(jnp.bfloat16)
    acc_ref[...] += jax.lax.dot_general(
        act, w2_ref[0].astype(jnp.bfloat16), (((1,), (1,)), ((), ())),
        preferred_element_type=jnp.float32)

    @pl.when(f == NF - 1)
    def _():
        ids_row = idr_ref[0]
        t_iota = jax.lax.broadcasted_iota(jnp.int32, (T, RB), 0)
        oht = (t_iota == ids_row).astype(jnp.bfloat16)
        z = (acc_ref[...] * gp_ref[0]).astype(jnp.bfloat16)
        out_ref[...] += jax.lax.dot_general(
            oht, z, (((1,), (0,)), ((), ())),
            preferred_element_type=jnp.float32)


def _moe_call(block_expert, ids_col, ids_row, gates_p, x2_bf16, w1b, w3b, w2b):
    grid_spec = pltpu.PrefetchScalarGridSpec(
        num_scalar_prefetch=1,
        grid=(NB, NF),
        in_specs=[
            pl.BlockSpec((1, RB, 1), lambda b, f, be: (b, 0, 0)),
            pl.BlockSpec((1, 1, RB), lambda b, f, be: (b, 0, 0)),
            pl.BlockSpec((1, RB, 1), lambda b, f, be: (b, 0, 0)),
            pl.BlockSpec((T, DM), lambda b, f, be: (0, 0)),
            pl.BlockSpec((1, FB, DM), lambda b, f, be: (be[b], f, 0)),
            pl.BlockSpec((1, FB, DM), lambda b, f, be: (be[b], f, 0)),
            pl.BlockSpec((1, DM, FB), lambda b, f, be: (be[b], 0, f)),
        ],
        out_specs=pl.BlockSpec((T, DM), lambda b, f, be: (0, 0)),
        scratch_shapes=[
            pltpu.VMEM((RB, DM), jnp.bfloat16),
            pltpu.VMEM((RB, DM), jnp.float32),
        ],
    )
    return pl.pallas_call(
        _moe_kernel,
        grid_spec=grid_spec,
        out_shape=jax.ShapeDtypeStruct((T, DM), jnp.float32),
    )(block_expert, ids_col, ids_row, gates_p, x2_bf16, w1b, w3b, w2b)


def _routing_metadata(topi, topv):
    P = NB * RB
    flat_e = topi.reshape(-1)
    order = jnp.argsort(flat_e, stable=True).astype(jnp.int32)
    e_iota = jnp.arange(E, dtype=jnp.int32)
    counts = jnp.sum(flat_e[:, None] == e_iota[None, :], axis=0,
                     dtype=jnp.int32)
    padded = ((counts + RB - 1) // RB) * RB
    cum_p = jnp.cumsum(padded)
    offs = cum_p - padded
    cum_c = jnp.cumsum(counts) - counts
    p_idx = jnp.arange(P, dtype=jnp.int32)
    eb_oh = (p_idx[:, None] >= cum_p[None, :])
    eb_p = jnp.minimum(jnp.sum(eb_oh, axis=1, dtype=jnp.int32), E - 1)
    sel = (eb_p[:, None] == e_iota[None, :])
    offs_p = jnp.sum(jnp.where(sel, offs[None, :], 0), axis=1)
    counts_p = jnp.sum(jnp.where(sel, counts[None, :], 0), axis=1)
    cum_c_p = jnp.sum(jnp.where(sel, cum_c[None, :], 0), axis=1)
    within = p_idx - offs_p
    valid = within < counts_p
    j = jnp.clip(cum_c_p + within, 0, T * TOPK - 1)
    oj = jnp.take(order, j)
    row_ids = jnp.where(valid, oj // TOPK, T)
    gates_p = jnp.where(valid, jnp.take(topv.reshape(-1), oj), 0.0)
    block_expert = eb_p[::RB]
    return (block_expert,
            row_ids.reshape(NB, RB, 1),
            row_ids.reshape(NB, 1, RB),
            gates_p.reshape(NB, RB, 1).astype(jnp.float32))


@jax.jit
def kernel(hidden_states, cos, sin, in_norm_w, post_norm_w, qkv_w, o_w,
           gate_w, w1, w3, w2):
    qkv = _qkv_call(hidden_states, in_norm_w.reshape(1, DM), qkv_w)
    ctx2d = _attn_call(qkv, cos, sin)
    attn_res, x2, topi, topv = _post_call(ctx2d, hidden_states, o_w,
                                          post_norm_w.reshape(1, DM), gate_w)
    block_expert, ids_col, ids_row, gates_p = _routing_metadata(topi, topv)
    moe_out = _moe_call(block_expert, ids_col, ids_row, gates_p,
                        x2.astype(jnp.bfloat16), w1, w3, w2)
    return (moe_out, attn_res)
